# Initial kernel scaffold; baseline (speedup 1.0000x reference)
import jax
import jax.numpy as jnp
from jax.experimental import pallas as pl


def kernel(x, edge_index, edge_attr, lin_e_W, lin_e_b, W1, b1, W2, b2, gamma, beta):
    raise NotImplementedError("write your pallas kernel here")


# speedup vs baseline: 3.0689x; 3.0689x over previous
import functools

import jax
import jax.numpy as jnp
from jax import lax
from jax.experimental import pallas as pl
from jax.experimental.pallas import tpu as pltpu
from jax.experimental.pallas import tpu_sc as plsc

N = 10000
E = 320000
D = 128
ED = 16
L = 4
BN_EPS = 1e-5
INV_STD = 1.0 / (1.0 + BN_EPS) ** 0.5

NC = 2
NS = 16
NW = NC * NS
EPW = E // NW
K = 80
NCHUNK = EPW // K
RPT = 632
NPAD = NS * RPT


def _edge_lin_body(ea_ref, w_ref, b_ref, o_ref):
    o_ref[...] = (
        jnp.dot(ea_ref[...], w_ref[...], preferred_element_type=jnp.float32)
        + b_ref[...]
    )


def _edge_linear(edge_attr, w, b):
    BE = 2000
    return pl.pallas_call(
        _edge_lin_body,
        grid=(E // BE,),
        in_specs=[
            pl.BlockSpec((BE, ED), lambda j: (j, 0)),
            pl.BlockSpec((ED, D), lambda j: (0, 0)),
            pl.BlockSpec((1, D), lambda j: (0, 0)),
        ],
        out_specs=pl.BlockSpec((BE, D), lambda j: (j, 0)),
        out_shape=jax.ShapeDtypeStruct((E, D), jnp.float32),
    )(edge_attr, w, b.reshape(1, D))


def _sc_body(h_hbm, e_hbm, src_hbm, dst_hbm, zero_hbm, out_hbm,
             src_v, dst_v, hrow_v, e_v, acc_sh, sem):
    cid = lax.axis_index("c")
    sid = lax.axis_index("s")
    wid = sid * NC + cid

    r0 = sid * RPT
    pltpu.sync_copy(zero_hbm.at[pl.ds(r0, RPT)], acc_sh.at[pl.ds(r0, RPT)])
    plsc.subcore_barrier()

    base0 = wid * EPW

    def chunk(ci, carry):
        b = base0 + ci * K
        pltpu.sync_copy(src_hbm.at[pl.ds(b, K)], src_v)
        gcp = pltpu.async_copy(h_hbm.at[src_v], hrow_v, sem)
        pltpu.sync_copy(e_hbm.at[pl.ds(b, K)], e_v)
        pltpu.sync_copy(dst_hbm.at[pl.ds(b, K)], dst_v)
        gcp.wait()

        def row(j, c2):
            for c in range(D // 16):
                sl = pl.ds(c * 16, 16)
                e_v[j, sl] = jnp.maximum(e_v[j, sl] + hrow_v[j, sl], 0.0)
            return c2

        lax.fori_loop(0, K, row, 0)
        pltpu.sync_copy(e_v, acc_sh.at[dst_v], add=True)
        return carry

    lax.fori_loop(0, NCHUNK, chunk, 0)
    plsc.subcore_barrier()
    pltpu.sync_copy(acc_sh.at[pl.ds(r0, RPT)], out_hbm.at[cid, pl.ds(r0, RPT)])


_sc_message = pl.kernel(
    _sc_body,
    out_type=jax.ShapeDtypeStruct((NC, NPAD, D), jnp.float32),
    mesh=plsc.VectorSubcoreMesh(core_axis_name="c", subcore_axis_name="s"),
    scratch_types=[
        pltpu.VMEM((K,), jnp.int32),
        pltpu.VMEM((K,), jnp.int32),
        pltpu.VMEM((K, D), jnp.float32),
        pltpu.VMEM((K, D), jnp.float32),
        pltpu.VMEM_SHARED((NPAD, D), jnp.float32),
        pltpu.SemaphoreType.DMA,
    ],
)


def _mlp_body(h_ref, a0_ref, a1_ref, w1_ref, b1_ref, w2_ref, b2_ref,
              g_ref, be_ref, o_ref):
    xb = h_ref[...] + a0_ref[...] + a1_ref[...]
    t = jnp.dot(xb, w1_ref[...], preferred_element_type=jnp.float32)
    t = jnp.maximum(t + b1_ref[...], 0.0)
    o = jnp.dot(t, w2_ref[...], preferred_element_type=jnp.float32) + b2_ref[...]
    o = o * (g_ref[...] * INV_STD) + be_ref[...]
    o_ref[...] = jnp.maximum(o, 0.0)


def _mlp(h, a0, a1, w1, b1, w2, b2, g, be):
    BNR = 1000
    row = pl.BlockSpec((BNR, D), lambda j: (j, 0))
    vec = pl.BlockSpec((1, D), lambda j: (0, 0))
    mat = pl.BlockSpec((D, D), lambda j: (0, 0))
    return pl.pallas_call(
        _mlp_body,
        grid=(N // BNR,),
        in_specs=[row, row, row, mat, vec, mat, vec, vec, vec],
        out_specs=row,
        out_shape=jax.ShapeDtypeStruct((N, D), jnp.float32),
    )(h, a0, a1, w1, b1.reshape(1, D), w2, b2.reshape(1, D),
      g.reshape(1, D), be.reshape(1, D))


@jax.jit
def _run(x, edge_index, edge_attr, lin_e_W, lin_e_b, W1, b1, W2, b2, gamma, beta):
    src = edge_index[0]
    dst = edge_index[1]
    zero = jnp.zeros((NPAD, D), jnp.float32)
    es = [_edge_linear(edge_attr, lin_e_W[i], lin_e_b[i]) for i in range(L)]
    h = x
    for i in range(L):
        parts = _sc_message(h, es[i], src, dst, zero)
        h = _mlp(h, parts[0, :N], parts[1, :N], W1[i], b1[i], W2[i], b2[i],
                 gamma[i], beta[i])
    return h


def kernel(x, edge_index, edge_attr, lin_e_W, lin_e_b, W1, b1, W2, b2, gamma, beta):
    return _run(x, edge_index, edge_attr, lin_e_W, lin_e_b, W1, b1, W2, b2,
                gamma, beta)


# speedup vs baseline: 5.2805x; 1.7206x over previous
import functools

import jax
import jax.numpy as jnp
from jax import lax
from jax.experimental import pallas as pl
from jax.experimental.pallas import tpu as pltpu
from jax.experimental.pallas import tpu_sc as plsc

N = 10000
E = 320000
D = 128
ED = 16
L = 4
BN_EPS = 1e-5
INV_STD = 1.0 / (1.0 + BN_EPS) ** 0.5

NC = 2
NS = 16
NW = NC * NS
EPW = E // NW
K = 80
NCHUNK = EPW // K
RPT = 632
NPAD = NS * RPT


def _edge_lin_body(ea_ref, w_ref, b_ref, o_ref):
    o_ref[...] = (
        jnp.dot(ea_ref[...], w_ref[...], preferred_element_type=jnp.float32)
        + b_ref[...]
    )


def _edge_linear(edge_attr, w, b):
    BE = 2000
    return pl.pallas_call(
        _edge_lin_body,
        grid=(E // BE,),
        in_specs=[
            pl.BlockSpec((BE, ED), lambda j: (j, 0)),
            pl.BlockSpec((ED, D), lambda j: (0, 0)),
            pl.BlockSpec((1, D), lambda j: (0, 0)),
        ],
        out_specs=pl.BlockSpec((BE, D), lambda j: (j, 0)),
        out_shape=jax.ShapeDtypeStruct((E, D), jnp.float32),
    )(edge_attr, w, b.reshape(1, D))


def _sc_body(h_hbm, e_hbm, sd_hbm, zero_hbm, out_hbm,
             i0, i1, i2, i3, h0, h1, e0, e1, acc_sh,
             zsem, is0, is1, is2, is3, gs0, gs1, es0, es1):
    cid = lax.axis_index("c")
    sid = lax.axis_index("s")
    wid = sid * NC + cid
    r0 = sid * RPT

    zcp = pltpu.async_copy(
        zero_hbm.at[pl.ds(r0, RPT)], acc_sh.at[pl.ds(r0, RPT)], zsem)

    ib = (i0, i1, i2, i3)
    isem = (is0, is1, is2, is3)
    hb = (h0, h1)
    eb = (e0, e1)
    gs = (gs0, gs1)
    es = (es0, es1)
    base0 = wid * EPW

    def start_idx(ci, q):
        cic = jnp.minimum(ci, NCHUNK - 1)
        pltpu.async_copy(sd_hbm.at[wid, cic], ib[q], isem[q])

    def wait_idx(q):
        pltpu.make_async_copy(sd_hbm.at[wid, 0], ib[q], isem[q]).wait()

    def start_ge(ci, q, p):
        wait_idx(q)
        pltpu.async_copy(h_hbm.at[ib[q].at[0]], hb[p], gs[p])
        pltpu.async_copy(e_hbm.at[pl.ds(base0 + ci * K, K)], eb[p], es[p])

    def finish(ci, q, p):
        pltpu.make_async_copy(h_hbm.at[ib[q].at[0]], hb[p], gs[p]).wait()
        pltpu.make_async_copy(
            e_hbm.at[pl.ds(base0 + ci * K, K)], eb[p], es[p]).wait()
        ev = eb[p]
        hv = hb[p]

        @plsc.parallel_loop(0, K)
        def row(j):
            for c in range(D // 16):
                sl = pl.ds(c * 16, 16)
                ev[j, sl] = jnp.maximum(ev[j, sl] + hv[j, sl], 0.0)

        pltpu.sync_copy(ev, acc_sh.at[ib[q].at[1]], add=True)

    start_idx(0, 0)
    start_idx(1, 1)
    zcp.wait()
    plsc.subcore_barrier()
    start_ge(0, 0, 0)

    def quad(o, carry):
        for b in range(4):
            ci = 4 * o + b
            start_idx(ci + 2, (b + 2) % 4)
            start_ge(ci + 1, (b + 1) % 4, (b + 1) % 2)
            finish(ci, b, b % 2)
        return carry

    lax.fori_loop(0, (NCHUNK - 1) // 4, quad, 0)
    wait_idx(1)
    finish(NCHUNK - 1, 0, 0)

    plsc.subcore_barrier()
    pltpu.sync_copy(acc_sh.at[pl.ds(r0, RPT)], out_hbm.at[cid, pl.ds(r0, RPT)])


_sc_message = pl.kernel(
    _sc_body,
    out_type=jax.ShapeDtypeStruct((NC, NPAD, D), jnp.float32),
    mesh=plsc.VectorSubcoreMesh(core_axis_name="c", subcore_axis_name="s"),
    scratch_types=[
        pltpu.VMEM((2, K), jnp.int32),
        pltpu.VMEM((2, K), jnp.int32),
        pltpu.VMEM((2, K), jnp.int32),
        pltpu.VMEM((2, K), jnp.int32),
        pltpu.VMEM((K, D), jnp.float32),
        pltpu.VMEM((K, D), jnp.float32),
        pltpu.VMEM((K, D), jnp.float32),
        pltpu.VMEM((K, D), jnp.float32),
        pltpu.VMEM_SHARED((NPAD, D), jnp.float32),
        pltpu.SemaphoreType.DMA,
        pltpu.SemaphoreType.DMA,
        pltpu.SemaphoreType.DMA,
        pltpu.SemaphoreType.DMA,
        pltpu.SemaphoreType.DMA,
        pltpu.SemaphoreType.DMA,
        pltpu.SemaphoreType.DMA,
        pltpu.SemaphoreType.DMA,
        pltpu.SemaphoreType.DMA,
    ],
)


def _mlp_body(h_ref, a0_ref, a1_ref, w1_ref, b1_ref, w2_ref, b2_ref,
              g_ref, be_ref, o_ref):
    xb = h_ref[...] + a0_ref[...] + a1_ref[...]
    t = jnp.dot(xb, w1_ref[...], preferred_element_type=jnp.float32)
    t = jnp.maximum(t + b1_ref[...], 0.0)
    o = jnp.dot(t, w2_ref[...], preferred_element_type=jnp.float32) + b2_ref[...]
    o = o * (g_ref[...] * INV_STD) + be_ref[...]
    o_ref[...] = jnp.maximum(o, 0.0)


def _mlp(h, a0, a1, w1, b1, w2, b2, g, be):
    BNR = 1000
    row = pl.BlockSpec((BNR, D), lambda j: (j, 0))
    vec = pl.BlockSpec((1, D), lambda j: (0, 0))
    mat = pl.BlockSpec((D, D), lambda j: (0, 0))
    return pl.pallas_call(
        _mlp_body,
        grid=(N // BNR,),
        in_specs=[row, row, row, mat, vec, mat, vec, vec, vec],
        out_specs=row,
        out_shape=jax.ShapeDtypeStruct((N, D), jnp.float32),
    )(h, a0, a1, w1, b1.reshape(1, D), w2, b2.reshape(1, D),
      g.reshape(1, D), be.reshape(1, D))


@jax.jit
def _run(x, edge_index, edge_attr, lin_e_W, lin_e_b, W1, b1, W2, b2, gamma, beta):
    sd = jnp.stack(
        [edge_index[0].reshape(NW, NCHUNK, K),
         edge_index[1].reshape(NW, NCHUNK, K)], axis=2)
    zero = jnp.zeros((NPAD, D), jnp.float32)
    es = [_edge_linear(edge_attr, lin_e_W[i], lin_e_b[i]) for i in range(L)]
    h = x
    for i in range(L):
        parts = _sc_message(h, es[i], sd, zero)
        h = _mlp(h, parts[0, :N], parts[1, :N], W1[i], b1[i], W2[i], b2[i],
                 gamma[i], beta[i])
    return h


def kernel(x, edge_index, edge_attr, lin_e_W, lin_e_b, W1, b1, W2, b2, gamma, beta):
    return _run(x, edge_index, edge_attr, lin_e_W, lin_e_b, W1, b1, W2, b2,
                gamma, beta)
